# baseline (speedup 1.0000x reference)
import functools

import jax
import jax.numpy as jnp
from jax import lax
from jax.experimental import pallas as pl
from jax.experimental.pallas import tpu as pltpu
from jax.experimental.pallas import tpu_sc as plsc

E = 1000000
D = 32
BROWS = 16384
S = 50
NC, NS = 2, 16
NW = NC * NS
BW = BROWS // NW
BWP = BW + 1
UNROLL = 8

_mesh = plsc.VectorSubcoreMesh(core_axis_name="c", subcore_axis_name="s")

CB = 512
E_MAIN = (E // 128) * 128
NCH = E_MAIN // CB
K_MAX = (NCH + NW - 1) // NW
CBP = CB + 1


@functools.partial(
    pl.kernel,
    mesh=_mesh,
    out_type=jax.ShapeDtypeStruct((E * D,), jnp.float32),
    scratch_types=[
        pltpu.VMEM((D, CB), jnp.float32),
        pltpu.VMEM((D, CB), jnp.float32),
        pltpu.VMEM((CB * D,), jnp.float32),
        pltpu.VMEM((CB * D,), jnp.float32),
        pltpu.VMEM(((E - E_MAIN) * D,), jnp.float32),
        pltpu.SemaphoreType.DMA,
        pltpu.SemaphoreType.DMA,
        pltpu.SemaphoreType.DMA,
        pltpu.SemaphoreType.DMA,
    ],
    compiler_params=pltpu.CompilerParams(
        use_tc_tiling_on_sc=True, needs_layout_passes=False),
)
def _detile_kernel(tableT_hbm, tail_hbm, out_hbm,
                   bufT0, bufT1, rowb0, rowb1, tailb, sr0, sr1, sw0, sw1):
    wid = lax.axis_index("s") * NC + lax.axis_index("c")
    bufT = (bufT0, bufT1)
    rowb = (rowb0, rowb1)
    sr = (sr0, sr1)
    sw = (sw0, sw1)

    iota16 = lax.iota(jnp.int32, 16)
    d_diag = [(iota16 + s) % D for s in range(D)]
    st_diag = [iota16 * D + ((iota16 + s) % D) for s in range(D)]

    @pl.when(wid == NW - 1)
    def _():
        pltpu.sync_copy(tail_hbm, tailb)
        pltpu.sync_copy(tailb, out_hbm.at[pl.ds(E_MAIN * D, (E - E_MAIN) * D)])

    def _read(k, b):
        c0 = (wid + k * NW) * CB
        pltpu.async_copy(tableT_hbm.at[:, pl.ds(c0, CB)], bufT[b], sr[b])

    def _read_wait(b):
        pltpu.make_async_copy(tableT_hbm.at[:, pl.ds(0, CB)],
                              bufT[b], sr[b]).wait()

    def _write(k, b):
        c0 = (wid + k * NW) * CB
        pltpu.async_copy(rowb[b], out_hbm.at[pl.ds(c0 * D, CB * D)], sw[b])

    def _write_wait(b):
        pltpu.make_async_copy(rowb[b], out_hbm.at[pl.ds(0, CB * D)],
                              sw[b]).wait()

    @pl.when(wid + 0 * NW < NCH)
    def _():
        _read(0, 0)

    @pl.when(wid + 1 * NW < NCH)
    def _():
        _read(1, 1)

    def body(i, _):
        for b in range(2):
            k = 2 * i + b
            c = wid + k * NW

            @pl.when(c < NCH)
            def _():
                _read_wait(b)

                @pl.when(k >= 2)
                def _():
                    _write_wait(b)

                @plsc.parallel_loop(0, CB // 16, 1, unroll=2)
                def _(g):
                    r_vec = iota16 + g * 16
                    base = g * (16 * D)
                    for s in range(D):
                        v = plsc.load_gather(bufT[b], [d_diag[s], r_vec])
                        plsc.store_scatter(rowb[b], [st_diag[s] + base], v)

                _write(k, b)

                @pl.when(wid + (k + 2) * NW < NCH)
                def _():
                    _read(k + 2, b)
        return 0

    lax.fori_loop(0, (K_MAX + 1) // 2, body, 0)

    for b in range(2):
        @pl.when(wid + b * NW < NCH)
        def _():
            _write_wait(b)


@functools.partial(
    pl.kernel,
    mesh=_mesh,
    out_type=jax.ShapeDtypeStruct((S, D, BROWS), jnp.float32),
    scratch_types=[
        pltpu.VMEM((S, BW), jnp.int32),
        pltpu.VMEM((BW, D), jnp.float32),
        pltpu.VMEM((BW, D), jnp.float32),
        pltpu.VMEM((D, BW), jnp.float32),
        pltpu.VMEM((D, BW), jnp.float32),
        pltpu.SemaphoreType.DMA,
        pltpu.SemaphoreType.DMA,
        pltpu.SemaphoreType.DMA,
        pltpu.SemaphoreType.DMA,
    ],
    compiler_params=pltpu.CompilerParams(
        use_tc_tiling_on_sc=False, needs_layout_passes=False),
)
def _gather_kernel(actionT_hbm, table_hbm, outT_hbm,
                   idx_v, rows0, rows1, rT0, rT1, sg0, sg1, sw0, sw1):
    wid = lax.axis_index("s") * NC + lax.axis_index("c")
    b0 = wid * BW
    rows = (rows0, rows1)
    rT = (rT0, rT1)
    sg = (sg0, sg1)
    sw = (sw0, sw1)

    iota16 = lax.iota(jnp.int32, 16)
    d_diag = [(iota16 + s) % D for s in range(D)]

    pltpu.sync_copy(actionT_hbm.at[:, pl.ds(b0, BW)], idx_v)

    pltpu.async_copy(table_hbm.at[idx_v.at[0]], rows0, sg0)
    pltpu.async_copy(table_hbm.at[idx_v.at[1]], rows1, sg1)

    def s_body(s2, _):
        for b in range(2):
            s = s2 * 2 + b
            pltpu.make_async_copy(
                table_hbm.at[idx_v.at[s]], rows[b], sg[b]).wait()
            @pl.when(s2 >= 1)
            def _():
                pltpu.make_async_copy(
                    rT[b],
                    outT_hbm.at[s, :, pl.ds(b0, BW)], sw[b]).wait()

            @plsc.parallel_loop(0, BW // 16, 1, unroll=2)
            def _(g):
                r_vec = iota16 + g * 16
                for sdiag in range(D):
                    v = plsc.load_gather(rows[b], [r_vec, d_diag[sdiag]])
                    plsc.store_scatter(rT[b], [d_diag[sdiag], r_vec], v)

            pltpu.async_copy(
                rT[b],
                outT_hbm.at[s, :, pl.ds(b0, BW)], sw[b])

            @pl.when(s + 2 < S)
            def _():
                pltpu.async_copy(
                    table_hbm.at[idx_v.at[s + 2]], rows[b], sg[b])
        return 0

    lax.fori_loop(0, S // 2, s_body, 0)

    pltpu.make_async_copy(
        rT0,
        outT_hbm.at[S - 2, :, pl.ds(b0, BW)], sw0).wait()
    pltpu.make_async_copy(
        rT1,
        outT_hbm.at[S - 1, :, pl.ds(b0, BW)], sw1).wait()


def kernel(action, emb_table):
    actionT = action.T
    tableT = emb_table.T
    tail = emb_table[E_MAIN:, :].reshape(-1)
    table_lin = _detile_kernel(tableT, tail)
    outT = _gather_kernel(actionT, table_lin.reshape(E, D))
    return outT.transpose(2, 0, 1)


# speedup vs baseline: 1.0374x; 1.0374x over previous
import functools

import jax
import jax.numpy as jnp
from jax import lax
from jax.experimental import pallas as pl
from jax.experimental.pallas import tpu as pltpu
from jax.experimental.pallas import tpu_sc as plsc

E = 1000000
D = 32
BROWS = 16384
S = 50
NC, NS = 2, 16
NW = NC * NS
BW = BROWS // NW
BWP = BW + 1
UNROLL = 8

_mesh = plsc.VectorSubcoreMesh(core_axis_name="c", subcore_axis_name="s")

CB = 512
E_MAIN = (E // 128) * 128
NCH = E_MAIN // CB
K_MAX = (NCH + NW - 1) // NW
CBP = CB + 1


@functools.partial(
    pl.kernel,
    mesh=_mesh,
    out_type=jax.ShapeDtypeStruct((E * D,), jnp.float32),
    scratch_types=[
        pltpu.VMEM((D, CB), jnp.float32),
        pltpu.VMEM((D, CB), jnp.float32),
        pltpu.VMEM((CB * D,), jnp.float32),
        pltpu.VMEM((CB * D,), jnp.float32),
        pltpu.VMEM(((E - E_MAIN) * D,), jnp.float32),
        pltpu.SemaphoreType.DMA,
        pltpu.SemaphoreType.DMA,
        pltpu.SemaphoreType.DMA,
        pltpu.SemaphoreType.DMA,
    ],
    compiler_params=pltpu.CompilerParams(
        use_tc_tiling_on_sc=True, needs_layout_passes=False),
)
def _detile_kernel(tableT_hbm, tail_hbm, out_hbm,
                   bufT0, bufT1, rowb0, rowb1, tailb, sr0, sr1, sw0, sw1):
    wid = lax.axis_index("s") * NC + lax.axis_index("c")
    bufT = (bufT0, bufT1)
    rowb = (rowb0, rowb1)
    sr = (sr0, sr1)
    sw = (sw0, sw1)

    iota16 = lax.iota(jnp.int32, 16)
    d_diag = [(iota16 + s) % D for s in range(D)]
    st_diag = [iota16 * D + ((iota16 + s) % D) for s in range(D)]

    @pl.when(wid == NW - 1)
    def _():
        pltpu.sync_copy(tail_hbm, tailb)
        pltpu.sync_copy(tailb, out_hbm.at[pl.ds(E_MAIN * D, (E - E_MAIN) * D)])

    def _read(k, b):
        c0 = (wid + k * NW) * CB
        pltpu.async_copy(tableT_hbm.at[:, pl.ds(c0, CB)], bufT[b], sr[b])

    def _read_wait(b):
        pltpu.make_async_copy(tableT_hbm.at[:, pl.ds(0, CB)],
                              bufT[b], sr[b]).wait()

    def _write(k, b):
        c0 = (wid + k * NW) * CB
        pltpu.async_copy(rowb[b], out_hbm.at[pl.ds(c0 * D, CB * D)], sw[b])

    def _write_wait(b):
        pltpu.make_async_copy(rowb[b], out_hbm.at[pl.ds(0, CB * D)],
                              sw[b]).wait()

    @pl.when(wid + 0 * NW < NCH)
    def _():
        _read(0, 0)

    @pl.when(wid + 1 * NW < NCH)
    def _():
        _read(1, 1)

    def body(i, _):
        for b in range(2):
            k = 2 * i + b
            c = wid + k * NW

            @pl.when(c < NCH)
            def _():
                _read_wait(b)

                @pl.when(k >= 2)
                def _():
                    _write_wait(b)

                @plsc.parallel_loop(0, CB // 16, 1, unroll=2)
                def _(g):
                    r_vec = iota16 + g * 16
                    base = g * (16 * D)
                    for s in range(D):
                        v = plsc.load_gather(bufT[b], [d_diag[s], r_vec])
                        plsc.store_scatter(rowb[b], [st_diag[s] + base], v)

                _write(k, b)

                @pl.when(wid + (k + 2) * NW < NCH)
                def _():
                    _read(k + 2, b)
        return 0

    lax.fori_loop(0, (K_MAX + 1) // 2, body, 0)

    for b in range(2):
        @pl.when(wid + b * NW < NCH)
        def _():
            _write_wait(b)


@functools.partial(
    pl.kernel,
    mesh=_mesh,
    out_type=jax.ShapeDtypeStruct((S, D, BROWS), jnp.float32),
    scratch_types=[
        pltpu.VMEM((S, BW), jnp.int32),
        pltpu.VMEM((BW, D), jnp.float32),
        pltpu.VMEM((BW, D), jnp.float32),
        pltpu.VMEM((D, BWP), jnp.float32),
        pltpu.VMEM((D, BWP), jnp.float32),
        pltpu.SemaphoreType.DMA,
        pltpu.SemaphoreType.DMA,
        pltpu.SemaphoreType.DMA,
        pltpu.SemaphoreType.DMA,
    ],
    compiler_params=pltpu.CompilerParams(
        use_tc_tiling_on_sc=False, needs_layout_passes=False),
)
def _gather_kernel(actionT_hbm, table_hbm, outT_hbm,
                   idx_v, rows0, rows1, rT0, rT1, sg0, sg1, sw0, sw1):
    wid = lax.axis_index("s") * NC + lax.axis_index("c")
    b0 = wid * BW
    rows = (rows0, rows1)
    rT = (rT0, rT1)
    sg = (sg0, sg1)
    sw = (sw0, sw1)

    iota16 = lax.iota(jnp.int32, 16)
    d_lo = iota16
    d_hi = iota16 + 16

    pltpu.sync_copy(actionT_hbm.at[:, pl.ds(b0, BW)], idx_v)

    pltpu.async_copy(table_hbm.at[idx_v.at[0]], rows0, sg0)
    pltpu.async_copy(table_hbm.at[idx_v.at[1]], rows1, sg1)

    def s_body(s2, _):
        for b in range(2):
            s = s2 * 2 + b
            pltpu.make_async_copy(
                table_hbm.at[idx_v.at[s]], rows[b], sg[b]).wait()
            @pl.when(s2 >= 1)
            def _():
                pltpu.make_async_copy(
                    rT[b].at[:, pl.ds(0, BW)],
                    outT_hbm.at[s, :, pl.ds(b0, BW)], sw[b]).wait()

            @plsc.parallel_loop(0, BW, 1, unroll=UNROLL)
            def _(r):
                r_vec = jnp.full((16,), 0, jnp.int32) + r
                lo = rows[b][r, pl.ds(0, 16)]
                hi = rows[b][r, pl.ds(16, 16)]
                plsc.store_scatter(rT[b], [d_lo, r_vec], lo)
                plsc.store_scatter(rT[b], [d_hi, r_vec], hi)

            pltpu.async_copy(
                rT[b].at[:, pl.ds(0, BW)],
                outT_hbm.at[s, :, pl.ds(b0, BW)], sw[b])

            @pl.when(s + 2 < S)
            def _():
                pltpu.async_copy(
                    table_hbm.at[idx_v.at[s + 2]], rows[b], sg[b])
        return 0

    lax.fori_loop(0, S // 2, s_body, 0)

    pltpu.make_async_copy(
        rT0.at[:, pl.ds(0, BW)],
        outT_hbm.at[S - 2, :, pl.ds(b0, BW)], sw0).wait()
    pltpu.make_async_copy(
        rT1.at[:, pl.ds(0, BW)],
        outT_hbm.at[S - 1, :, pl.ds(b0, BW)], sw1).wait()


def kernel(action, emb_table):
    actionT = action.T
    tableT = emb_table.T
    tail = emb_table[E_MAIN:, :].reshape(-1)
    table_lin = _detile_kernel(tableT, tail)
    outT = _gather_kernel(actionT, table_lin.reshape(E, D))
    return outT.transpose(2, 0, 1)
